# baseline (speedup 1.0000x reference)
import functools

import jax
import jax.numpy as jnp
from jax import lax
from jax.experimental import pallas as pl
from jax.experimental.pallas import tpu as pltpu
from jax.experimental.pallas import tpu_sc as plsc

NC = 2
NS = 16
L = 16
NW = NC * NS

B = 16384
D = 64
V = 1000000

SLABW = 512
NSLAB = V // SLABW + 1
TAILS = V // SLABW
TAILW = V - TAILS * SLABW
NT = (NSLAB + NW - 1) // NW
NBUF = 3
NROUND = (NT + NBUF - 1) // NBUF

HITMAX = 768
BUCKMAX = 192
SSTRIP = 1024
NSTRIP = B // SSTRIP
OUTW = 128
RING = 128
FCHUNK = 32
DUMMY = B

NEG_SLOPE = 0.01


def kernel(all_id, table):
    table_t = table.T
    tail_t = jnp.pad(table_t[:, TAILS * SLABW:], ((0, 0), (0, 128 - TAILW)))
    mesh = plsc.VectorSubcoreMesh(core_axis_name="c", subcore_axis_name="s")

    @functools.partial(
        pl.kernel,
        out_type=jax.ShapeDtypeStruct((B + 8, OUTW), jnp.float32),
        mesh=mesh,
        compiler_params=pltpu.CompilerParams(needs_layout_passes=False),
        scratch_types=(
            [
                pltpu.VMEM((SSTRIP,), jnp.int32),
                pltpu.VMEM((SSTRIP,), jnp.int32),
                pltpu.VMEM((HITMAX + L,), jnp.int32),
                pltpu.VMEM((8, BUCKMAX + L), jnp.int32),
                pltpu.VMEM((128 + L,), jnp.int32),
                pltpu.VMEM((D, SLABW), jnp.float32),
                pltpu.VMEM((D, SLABW), jnp.float32),
                pltpu.VMEM((D, SLABW), jnp.float32),
                pltpu.VMEM((RING, OUTW), jnp.float32),
                pltpu.VMEM((RING // FCHUNK, FCHUNK), jnp.int32),
                pltpu.SMEM((8,), jnp.int32),
            ]
            + [pltpu.SemaphoreType.DMA for _ in range(6)]
        ),
    )
    def k(idx_hbm, table_hbm, tail_hbm, out_hbm, strip0, strip1, hitbuf,
          buckets, slabhits, slab0, slab1, slab2, ring, oidx, bcnt_s,
          ss0, ss1, sg0, sg1, sg2, fsem):
        strips = (strip0, strip1)
        ssems = (ss0, ss1)
        slabs = (slab0, slab1, slab2)
        gsems = (sg0, sg1, sg2)

        wid = lax.axis_index("s") * NC + lax.axis_index("c")
        lanes = lax.iota(jnp.int32, L)
        lanes9 = lanes << 9

        def fire_slab(t, tbuf):
            sid = wid + NW * t
            buf = slabs[tbuf]
            sem = gsems[tbuf]

            @pl.when(sid < TAILS)
            def _():
                pltpu.async_copy(
                    table_hbm.at[pl.ds(0, D), pl.ds(sid * SLABW, SLABW)],
                    buf, sem,
                )

            @pl.when(sid == TAILS)
            def _():
                pltpu.async_copy(
                    tail_hbm, buf.at[pl.ds(0, D), pl.ds(0, 128)], sem
                )

        def wait_slab(t, tbuf):
            sid = wid + NW * t
            buf = slabs[tbuf]
            sem = gsems[tbuf]

            @pl.when(sid < TAILS)
            def _():
                pltpu.make_async_copy(
                    table_hbm.at[pl.ds(0, D), pl.ds(0, SLABW)], buf, sem
                ).wait()

            @pl.when(sid == TAILS)
            def _():
                pltpu.make_async_copy(
                    tail_hbm, buf.at[pl.ds(0, D), pl.ds(0, 128)], sem
                ).wait()

        fire_slab(0, 0)
        fire_slab(1, 1)
        fire_slab(2, 2)

        hcnt = jnp.int32(0)
        pltpu.async_copy(idx_hbm.at[pl.ds(0, SSTRIP)], strip0, ss0)
        for s in range(NSTRIP):
            if s + 1 < NSTRIP:
                pltpu.async_copy(
                    idx_hbm.at[pl.ds((s + 1) * SSTRIP, SSTRIP)],
                    strips[(s + 1) % 2], ssems[(s + 1) % 2],
                )
            pltpu.make_async_copy(
                idx_hbm.at[pl.ds(0, SSTRIP)], strips[s % 2], ssems[s % 2]
            ).wait()
            strip = strips[s % 2]

            def scan_body(g, cnt, s=s, strip=strip):
                vec = strip[pl.ds(g * L, L)]
                mine = ((vec >> 9) & (NW - 1)) == wid
                pack = (
                    ((vec >> 14) << 23)
                    | (((s * SSTRIP + g * L) << 9) | lanes9)
                    | (vec & (SLABW - 1))
                )
                plsc.store_compressed(
                    hitbuf.at[pl.ds(cnt, L)], pack, mask=mine
                )
                return cnt + jnp.sum(jnp.where(mine, 1, 0))

            hcnt = lax.fori_loop(0, SSTRIP // L, scan_body, hcnt)

        def bucket_body(g, bcs):
            e = hitbuf[pl.ds(g * L, L)]
            sbv = e >> 26
            inb = g * L + lanes < hcnt
            out = []
            for sb in range(8):
                m = (sbv == sb) & inb
                plsc.store_compressed(
                    buckets.at[sb, pl.ds(bcs[sb], L)], e, mask=m
                )
                out.append(bcs[sb] + jnp.sum(jnp.where(m, 1, 0)))
            return tuple(out)

        bcs = lax.fori_loop(
            0, (hcnt + L - 1) // L, bucket_body,
            tuple(jnp.int32(0) for _ in range(8)),
        )
        for sb in range(8):
            bcnt_s[sb] = bcs[sb]

        def flush_ring(fl, pend):
            @pl.when(pend > 0)
            def _():
                pltpu.make_async_copy(
                    ring.at[pl.ds(0, FCHUNK)], out_hbm.at[oidx.at[0]], fsem
                ).wait()

            half = (fl >> 5) & (RING // FCHUNK - 1)
            pltpu.async_copy(
                ring.at[pl.ds(fl & (RING - 1), FCHUNK)],
                out_hbm.at[oidx.at[half]], fsem,
            )
            return fl + FCHUNK, jnp.int32(1)

        def process_slab(t, tbuf, carry):
            oc, fl, pend = carry
            buf = slabs[tbuf]
            sb = t >> 3
            bc = bcnt_s[sb]

            def filter_body(g, sc):
                e = buckets[sb, pl.ds(g * L, L)]
                m = ((e >> 23) == t) & (g * L + lanes < bc)
                plsc.store_compressed(
                    slabhits.at[pl.ds(sc, L)], e & 0x7FFFFF, mask=m
                )
                return sc + jnp.sum(jnp.where(m, 1, 0))

            sc = lax.fori_loop(0, (bc + L - 1) // L, filter_body,
                               jnp.int32(0))
            wait_slab(t, tbuf)

            def extract_body(h, carry):
                oc, fl, pend = carry
                do_flush = oc - fl >= FCHUNK
                fl2, pend2 = lax.cond(
                    do_flush, flush_ring, lambda a, b: (a, b), fl, pend
                )
                e = slabhits[pl.ds(h * L, L)]
                rem = jnp.minimum(sc - h * L, L)
                valid = lanes < rem
                kvec = (e >> 9) & (B - 1)
                cvec = e & (SLABW - 1)
                pos = oc + lanes
                plsc.store_scatter(
                    oidx, [(pos >> 5) & (RING // FCHUNK - 1),
                           pos & (FCHUNK - 1)], kvec, mask=valid,
                )
                rows = pos & (RING - 1)
                for d in range(D):
                    dsplat = jnp.full((L,), d, jnp.int32)
                    v = plsc.load_gather(buf, [dsplat, cvec], mask=valid)
                    v = jnp.maximum(v, v * NEG_SLOPE)
                    plsc.store_scatter(ring, [rows, dsplat], v, mask=valid)
                return oc + rem, fl2, pend2

            return lax.fori_loop(0, (sc + L - 1) // L, extract_body,
                                 (oc, fl, pend))

        def round_body(u, carry):
            t0 = NBUF * u
            for b in range(NBUF):
                carry = process_slab(t0 + b, b, carry)
                fire_slab(t0 + b + NBUF, b)
            return carry

        oc, fl, pend = lax.fori_loop(
            0, NROUND, round_body,
            (jnp.int32(0), jnp.int32(0), jnp.int32(0)),
        )

        npad = (-oc) & (FCHUNK - 1)
        for i in range(FCHUNK // L):
            pos = oc + i * L + lanes
            valid = pos < oc + npad
            plsc.store_scatter(
                oidx, [(pos >> 5) & (RING // FCHUNK - 1), pos & (FCHUNK - 1)],
                jnp.full((L,), DUMMY, jnp.int32), mask=valid,
            )
        for _i in range(2):
            fl, pend = lax.cond(
                oc + npad - fl >= FCHUNK, flush_ring,
                lambda a, b: (a, b), fl, pend,
            )

        @pl.when(pend > 0)
        def _():
            pltpu.make_async_copy(
                ring.at[pl.ds(0, FCHUNK)], out_hbm.at[oidx.at[0]], fsem
            ).wait()

    out = k(all_id, table_t, tail_t)
    return out[:B, :D]
